# baseline (speedup 1.0000x reference)
import functools

import jax
import jax.numpy as jnp
from jax.experimental import pallas as pl
from jax.experimental.pallas import tpu as pltpu

_BS = 1024
_BB = 16
_NSEM = 8


def _tc_broadcast(table):
    num_mode, d_model = table.shape
    n_chunks = _BS // _BB
    rows = _BB * num_mode

    def body(in_ref, out_ref, stage, sem_in, sem_out):
        out2d = out_ref.reshape(_BS * num_mode, d_model)
        pltpu.make_async_copy(in_ref, stage.at[pl.ds(0, num_mode)],
                              sem_in).start()
        pltpu.make_async_copy(in_ref, stage.at[pl.ds(0, num_mode)],
                              sem_in).wait()
        s3 = stage.reshape(_BB, num_mode, d_model)
        s3[...] = jnp.broadcast_to(s3[pl.ds(0, 1)], (_BB, num_mode, d_model))
        for i in range(n_chunks):
            pltpu.make_async_copy(
                stage, out2d.at[pl.ds(i * rows, rows)],
                sem_out.at[i % _NSEM]).start()
        for i in range(n_chunks):
            pltpu.make_async_copy(
                stage, out2d.at[pl.ds(i * rows, rows)],
                sem_out.at[i % _NSEM]).wait()

    return pl.pallas_call(
        body,
        in_specs=[pl.BlockSpec(memory_space=pltpu.HBM)],
        out_specs=pl.BlockSpec(memory_space=pltpu.HBM),
        out_shape=jax.ShapeDtypeStruct((_BS, num_mode, d_model), jnp.float32),
        scratch_shapes=[
            pltpu.VMEM((rows, d_model), jnp.float32),
            pltpu.SemaphoreType.DMA,
            pltpu.SemaphoreType.DMA((_NSEM,)),
        ],
    )(table)


def kernel(mode_emb_weight, bs, num_mode):
    del bs, num_mode
    return _tc_broadcast(mode_emb_weight)
